# baseline (speedup 1.0000x reference)
import functools

import jax
import jax.numpy as jnp
from jax.experimental import pallas as pl
from jax.experimental.pallas import tpu as pltpu


def _round_up(a, b):
    return ((a + b - 1) // b) * b


def _gc_kernel(x_ref, nbr_ref, w_ref, e_ref, agg_ref, out_ref, xw_ref):
    tb, kb, fin = nbr_ref.shape
    fout = w_ref.shape[1]
    j = pl.program_id(1)
    w = w_ref[...]
    e = e_ref[...]

    @pl.when(j == 0)
    def _():
        xw_ref[...] = jnp.dot(x_ref[...], w, preferred_element_type=jnp.float32)

    nbr2 = nbr_ref[...].reshape(tb * kb, fin)
    nw = jnp.dot(nbr2, w, preferred_element_type=jnp.float32)
    xw_rep = jnp.dot(e, xw_ref[...], preferred_element_type=jnp.float32)
    out_ref[...] = (nw + xw_rep).reshape(tb, kb, fout).astype(out_ref.dtype)

    agg = jax.lax.dot_general(
        e, nw, (((0,), (0,)), ((), ())), preferred_element_type=jnp.float32
    )

    @pl.when(j == 0)
    def _():
        agg_ref[...] = agg.astype(agg_ref.dtype)

    @pl.when(j > 0)
    def _():
        agg_ref[...] += agg.astype(agg_ref.dtype)


@functools.partial(jax.jit, static_argnames=("block_b", "block_k"))
def _graph_conv(x, neighbor, weight_t, block_b=512, block_k=8):
    B, Fin = x.shape
    _, K, _ = neighbor.shape
    Fout = weight_t.shape[1]

    block_b = min(block_b, _round_up(B, 8))
    B_pad = _round_up(B, block_b)
    if B_pad > B:
        x = jnp.pad(x, ((0, B_pad - B), (0, 0)))
        neighbor = jnp.pad(neighbor, ((0, B_pad - B), (0, 0), (0, 0)))

    e = jnp.repeat(jnp.eye(block_b, dtype=x.dtype), block_k, axis=0)

    grid = (B_pad // block_b, K // block_k)

    aggred, neighbor_out = pl.pallas_call(
        _gc_kernel,
        out_shape=(
            jax.ShapeDtypeStruct((B_pad, Fout), x.dtype),
            jax.ShapeDtypeStruct((B_pad, K, Fout), x.dtype),
        ),
        grid=grid,
        in_specs=[
            pl.BlockSpec((block_b, Fin), lambda i, j: (i, 0)),
            pl.BlockSpec((block_b, block_k, Fin), lambda i, j: (i, j, 0)),
            pl.BlockSpec((Fin, Fout), lambda i, j: (0, 0)),
            pl.BlockSpec((block_b * block_k, block_b), lambda i, j: (0, 0)),
        ],
        out_specs=(
            pl.BlockSpec((block_b, Fout), lambda i, j: (i, 0)),
            pl.BlockSpec((block_b, block_k, Fout), lambda i, j: (i, j, 0)),
        ),
        scratch_shapes=[pltpu.VMEM((block_b, Fout), jnp.float32)],
        compiler_params=pltpu.CompilerParams(
            dimension_semantics=("parallel", "arbitrary"),
            vmem_limit_bytes=48 * 1024 * 1024,
        ),
    )(x, neighbor, weight_t, e)

    return aggred[:B], neighbor_out[:B]


def kernel(x, neighbor, weight_t):
    return _graph_conv(x, neighbor, weight_t)


# speedup vs baseline: 1.3430x; 1.3430x over previous
import functools

import jax
import jax.numpy as jnp
from jax.experimental import pallas as pl
from jax.experimental.pallas import tpu as pltpu


def _round_up(a, b):
    return ((a + b - 1) // b) * b


def _gc_kernel(x_ref, nbr_ref, w_ref, e_ref, agg_ref, out_ref):
    tb, k, fin = nbr_ref.shape
    fout = w_ref.shape[1]
    w = w_ref[...]
    e = e_ref[...]

    nbr2 = nbr_ref[...].reshape(tb * k, fin)
    nw = jnp.dot(nbr2, w, preferred_element_type=jnp.float32)

    xw = jnp.dot(x_ref[...], w, preferred_element_type=jnp.float32)
    xw_rep = jnp.dot(e, xw, preferred_element_type=jnp.float32)

    out_ref[...] = (nw + xw_rep).reshape(tb, k, fout).astype(out_ref.dtype)

    agg = jax.lax.dot_general(
        e, nw, (((0,), (0,)), ((), ())), preferred_element_type=jnp.float32
    )
    agg_ref[...] = agg.astype(agg_ref.dtype)


@functools.partial(jax.jit, static_argnames=("block_b",))
def _graph_conv(x, neighbor, weight_t, block_b=256):
    B, Fin = x.shape
    _, K, _ = neighbor.shape
    Fout = weight_t.shape[1]

    block_b = min(block_b, _round_up(B, 8))
    B_pad = _round_up(B, block_b)
    if B_pad > B:
        x = jnp.pad(x, ((0, B_pad - B), (0, 0)))
        neighbor = jnp.pad(neighbor, ((0, B_pad - B), (0, 0), (0, 0)))

    e = jnp.repeat(jnp.eye(block_b, dtype=x.dtype), K, axis=0)

    grid = (B_pad // block_b,)

    aggred, neighbor_out = pl.pallas_call(
        _gc_kernel,
        out_shape=(
            jax.ShapeDtypeStruct((B_pad, Fout), x.dtype),
            jax.ShapeDtypeStruct((B_pad, K, Fout), x.dtype),
        ),
        grid=grid,
        in_specs=[
            pl.BlockSpec((block_b, Fin), lambda i: (i, 0)),
            pl.BlockSpec((block_b, K, Fin), lambda i: (i, 0, 0)),
            pl.BlockSpec((Fin, Fout), lambda i: (0, 0)),
            pl.BlockSpec((block_b * K, block_b), lambda i: (0, 0)),
        ],
        out_specs=(
            pl.BlockSpec((block_b, Fout), lambda i: (i, 0)),
            pl.BlockSpec((block_b, K, Fout), lambda i: (i, 0, 0)),
        ),
        compiler_params=pltpu.CompilerParams(
            dimension_semantics=("parallel",),
            vmem_limit_bytes=48 * 1024 * 1024,
        ),
    )(x, neighbor, weight_t, e)

    return aggred[:B], neighbor_out[:B]


def kernel(x, neighbor, weight_t):
    return _graph_conv(x, neighbor, weight_t)
